# Initial kernel scaffold; baseline (speedup 1.0000x reference)
import functools

import jax
import jax.numpy as jnp
from jax import lax
from jax.experimental import pallas as pl
from jax.experimental.pallas import tpu as pltpu
from jax.experimental.pallas import tpu_sc as plsc

N = 10000
E = 320000
D = 128

NC = 2
NS = 16
NW = NC * NS

CH = 128
CHUNKS = E // CH
BASE_CH = CHUNKS // NW
EXTRA = CHUNKS - BASE_CH * NW
ROWS_PER_TILE = N // NS
CNT_W = 8

_mesh = plsc.VectorSubcoreMesh(core_axis_name="c", subcore_axis_name="s")


@functools.partial(
    pl.kernel,
    out_type=(
        jax.ShapeDtypeStruct((NC, N, D), jnp.float32),
        jax.ShapeDtypeStruct((NC, N, CNT_W), jnp.float32),
    ),
    mesh=_mesh,
    scratch_types=[
        pltpu.VMEM((2, CH), jnp.int32),
        pltpu.VMEM((2, CH), jnp.int32),
        pltpu.VMEM((2, CH, D), jnp.float32),
        pltpu.VMEM((CH, CNT_W), jnp.float32),
        pltpu.VMEM_SHARED((N, D), jnp.float32),
        pltpu.VMEM_SHARED((N, CNT_W), jnp.float32),
        pltpu.SemaphoreType.DMA,
    ],
)
def _sc_aggregate(src_hbm, dst_hbm, x_hbm, zacc_hbm, zcnt_hbm, ones_hbm,
                  out_sum, out_cnt,
                  srcv, dstv, rows, ones_v, acc_sh, cnt_sh, sem):
    c = lax.axis_index("c")
    s = lax.axis_index("s")
    wid = s * NC + c

    r0 = s * ROWS_PER_TILE
    pltpu.sync_copy(zacc_hbm.at[pl.ds(r0, ROWS_PER_TILE)],
                    acc_sh.at[pl.ds(r0, ROWS_PER_TILE)])
    pltpu.sync_copy(zcnt_hbm.at[pl.ds(r0, ROWS_PER_TILE)],
                    cnt_sh.at[pl.ds(r0, ROWS_PER_TILE)])
    pltpu.sync_copy(ones_hbm, ones_v)
    plsc.subcore_barrier()

    nch = BASE_CH + jnp.where(wid < EXTRA, 1, 0)

    def body(k, carry):
        off = (wid + k * NW) * CH
        pltpu.sync_copy(src_hbm.at[pl.ds(off, CH)], srcv.at[0])
        pltpu.sync_copy(dst_hbm.at[pl.ds(off, CH)], dstv.at[0])
        pltpu.async_copy(x_hbm.at[srcv.at[0]], rows.at[0], sem).wait()
        pltpu.sync_copy(rows.at[0], acc_sh.at[dstv.at[0]], add=True)
        pltpu.sync_copy(ones_v, cnt_sh.at[dstv.at[0]], add=True)
        return carry

    lax.fori_loop(0, nch, body, 0)
    plsc.subcore_barrier()

    pltpu.sync_copy(acc_sh.at[pl.ds(r0, ROWS_PER_TILE)],
                    out_sum.at[c, pl.ds(r0, ROWS_PER_TILE)])
    pltpu.sync_copy(cnt_sh.at[pl.ds(r0, ROWS_PER_TILE)],
                    out_cnt.at[c, pl.ds(r0, ROWS_PER_TILE)])


BLK = 400


def _tc_epilogue(ps_ref, cnt_ref, x_ref, wl_ref, wr_ref, bl_ref, o_ref):
    p = ps_ref[0] + ps_ref[1]
    cnt = (cnt_ref[0] + cnt_ref[1])[:, :1]
    mean = p / jnp.clip(cnt, 1.0, None)
    acc = jnp.dot(mean, wl_ref[...], preferred_element_type=jnp.float32)
    acc = acc + jnp.dot(x_ref[...], wr_ref[...],
                        preferred_element_type=jnp.float32)
    o_ref[...] = jnp.maximum(acc + bl_ref[...], 0.0)


def kernel(x, edge_index, W_l, W_r, b_l):
    dst = edge_index[0].astype(jnp.int32)
    src = edge_index[1].astype(jnp.int32)
    zacc = jnp.zeros((N, D), jnp.float32)
    zcnt = jnp.zeros((N, CNT_W), jnp.float32)
    ones = jnp.ones((CH, CNT_W), jnp.float32)

    psum, pcnt = _sc_aggregate(src, dst, x, zacc, zcnt, ones)

    out = pl.pallas_call(
        _tc_epilogue,
        grid=(N // BLK,),
        in_specs=[
            pl.BlockSpec((NC, BLK, D), lambda i: (0, i, 0)),
            pl.BlockSpec((NC, BLK, CNT_W), lambda i: (0, i, 0)),
            pl.BlockSpec((BLK, D), lambda i: (i, 0)),
            pl.BlockSpec((D, D), lambda i: (0, 0)),
            pl.BlockSpec((D, D), lambda i: (0, 0)),
            pl.BlockSpec((1, D), lambda i: (0, 0)),
        ],
        out_specs=pl.BlockSpec((BLK, D), lambda i: (i, 0)),
        out_shape=jax.ShapeDtypeStruct((N, D), jnp.float32),
    )(psum, pcnt, x, W_l, W_r, b_l.reshape(1, D))
    return out


# speedup vs baseline: 5.6354x; 5.6354x over previous
import functools

import jax
import jax.numpy as jnp
from jax import lax
from jax.experimental import pallas as pl
from jax.experimental.pallas import tpu as pltpu
from jax.experimental.pallas import tpu_sc as plsc

N = 10000
E = 320000
D = 128

NC = 2
NS = 16
NW = NC * NS

CH = 128
CHUNKS = E // CH
BASE_CH = CHUNKS // NW
EXTRA = CHUNKS - BASE_CH * NW
N_PAD = 10240
ROWS_PER_TILE = N_PAD // NS

_mesh = plsc.VectorSubcoreMesh(core_axis_name="c", subcore_axis_name="s")


@functools.partial(
    pl.kernel,
    out_type=jax.ShapeDtypeStruct((NC, N_PAD, D), jnp.float32),
    mesh=_mesh,
    scratch_types=[
        pltpu.VMEM((1, CH), jnp.int32),
        pltpu.VMEM((1, CH), jnp.int32),
        pltpu.VMEM((1, CH, D), jnp.float32),
        pltpu.VMEM_SHARED((N_PAD, D), jnp.float32),
        pltpu.SemaphoreType.DMA,
    ],
)
def _sc_sum(src_hbm, dst_hbm, x_hbm, zrow_hbm, out_sum,
            srcv, dstv, rows, acc_sh, sem):
    c = lax.axis_index("c")
    s = lax.axis_index("s")
    wid = s * NC + c
    r0 = s * ROWS_PER_TILE

    pltpu.sync_copy(zrow_hbm, rows.at[0])
    for k in range(ROWS_PER_TILE // CH):
        pltpu.sync_copy(rows.at[0], acc_sh.at[pl.ds(r0 + k * CH, CH)])
    plsc.subcore_barrier()

    nch = BASE_CH + jnp.where(wid < EXTRA, 1, 0)

    def body(k, carry):
        off = (wid + k * NW) * CH
        pltpu.sync_copy(src_hbm.at[pl.ds(off, CH)], srcv.at[0])
        pltpu.sync_copy(dst_hbm.at[pl.ds(off, CH)], dstv.at[0])
        pltpu.async_copy(x_hbm.at[srcv.at[0]], rows.at[0], sem).wait()
        pltpu.sync_copy(rows.at[0], acc_sh.at[dstv.at[0]], add=True)
        return carry

    lax.fori_loop(0, nch, body, 0)
    plsc.subcore_barrier()

    for k in range(ROWS_PER_TILE // CH):
        pltpu.sync_copy(acc_sh.at[pl.ds(r0 + k * CH, CH)], rows.at[0])
        pltpu.sync_copy(rows.at[0], out_sum.at[c, pl.ds(r0 + k * CH, CH)])


@functools.partial(
    pl.kernel,
    out_type=jax.ShapeDtypeStruct((NC, N_PAD, D), jnp.float32),
    mesh=_mesh,
    scratch_types=[
        pltpu.VMEM((1, CH), jnp.int32),
        pltpu.VMEM((CH, D), jnp.float32),
        pltpu.VMEM_SHARED((N_PAD, D), jnp.float32),
    ],
)
def _sc_count(dst_hbm, zrow_hbm, ones_hbm, out_cnt, dstv, ones_v, cnt_sh):
    c = lax.axis_index("c")
    s = lax.axis_index("s")
    wid = s * NC + c
    r0 = s * ROWS_PER_TILE

    pltpu.sync_copy(zrow_hbm, ones_v)
    for k in range(ROWS_PER_TILE // CH):
        pltpu.sync_copy(ones_v, cnt_sh.at[pl.ds(r0 + k * CH, CH)])
    pltpu.sync_copy(ones_hbm, ones_v)
    plsc.subcore_barrier()

    nch = BASE_CH + jnp.where(wid < EXTRA, 1, 0)

    def body(k, carry):
        off = (wid + k * NW) * CH
        pltpu.sync_copy(dst_hbm.at[pl.ds(off, CH)], dstv.at[0])
        pltpu.sync_copy(ones_v, cnt_sh.at[dstv.at[0]], add=True)
        return carry

    lax.fori_loop(0, nch, body, 0)
    plsc.subcore_barrier()

    for k in range(ROWS_PER_TILE // CH):
        pltpu.sync_copy(cnt_sh.at[pl.ds(r0 + k * CH, CH)], ones_v)
        pltpu.sync_copy(ones_v, out_cnt.at[c, pl.ds(r0 + k * CH, CH)])


BLK = 400


def _tc_epilogue(ps_ref, cnt_ref, x_ref, wl_ref, wr_ref, bl_ref, o_ref):
    p = ps_ref[0] + ps_ref[1]
    cnt = (cnt_ref[0] + cnt_ref[1])[:, :1]
    mean = p / jnp.clip(cnt, 1.0, None)
    acc = jnp.dot(mean, wl_ref[...], preferred_element_type=jnp.float32)
    acc = acc + jnp.dot(x_ref[...], wr_ref[...],
                        preferred_element_type=jnp.float32)
    o_ref[...] = jnp.maximum(acc + bl_ref[...], 0.0)


def kernel(x, edge_index, W_l, W_r, b_l):
    dst = edge_index[0].astype(jnp.int32)
    src = edge_index[1].astype(jnp.int32)
    zrow = jnp.zeros((CH, D), jnp.float32)
    ones = jnp.ones((CH, D), jnp.float32)

    psum = _sc_sum(src, dst, x, zrow)
    pcnt = _sc_count(dst, zrow, ones)

    out = pl.pallas_call(
        _tc_epilogue,
        grid=(N // BLK,),
        in_specs=[
            pl.BlockSpec((NC, BLK, D), lambda i: (0, i, 0)),
            pl.BlockSpec((NC, BLK, D), lambda i: (0, i, 0)),
            pl.BlockSpec((BLK, D), lambda i: (i, 0)),
            pl.BlockSpec((D, D), lambda i: (0, 0)),
            pl.BlockSpec((D, D), lambda i: (0, 0)),
            pl.BlockSpec((1, D), lambda i: (0, 0)),
        ],
        out_specs=pl.BlockSpec((BLK, D), lambda i: (i, 0)),
        out_shape=jax.ShapeDtypeStruct((N, D), jnp.float32),
    )(psum, pcnt, x, W_l, W_r, b_l.reshape(1, D))
    return out
